# baseline (speedup 1.0000x reference)
import functools
import math

import jax
import jax.numpy as jnp
from jax import lax
from jax.experimental import pallas as pl
from jax.experimental.pallas import tpu as pltpu
from jax.experimental.pallas import tpu_sc as plsc

B, C, N, K, G = 2, 128, 4096, 9, 4
CG = C // G
KG = K * G
NB_KNN = 256
NB_ATT = 512
NW = 32
ROWS_PER_W = B * N * K // NW
GCHUNK = 128
NCHUNK = ROWS_PER_W // GCHUNK


def _point_kernel(x_ref, wattn_ref, battn_ref, waw2_ref, baw2_ref,
                  u_ref, vt_ref, awp_ref):
    x = x_ref[0]
    w1 = wattn_ref[:, :C]
    w2 = wattn_ref[:, C:]
    wu = w1 + w2
    u_ref[0] = lax.dot_general(wu, x, (((1,), (0,)), ((), ())),
                               preferred_element_type=jnp.float32) \
        + battn_ref[:]
    vt_ref[0] = lax.dot_general(x, w2, (((0,), (1,)), ((), ())),
                                preferred_element_type=jnp.float32)
    awl = lax.dot_general(waw2_ref[:], x, (((0,), (0,)), ((), ())),
                          preferred_element_type=jnp.float32)
    awl = awl + baw2_ref[:]
    a3 = awl.reshape(K, G, N)
    m = jnp.max(a3, axis=0, keepdims=True)
    e = jnp.exp(a3 - m)
    p = e / jnp.sum(e, axis=0, keepdims=True)
    awp_ref[0] = p.reshape(KG, N)


def _point_call(x, w_attn, b_attn, w_aw2, b_aw2):
    return pl.pallas_call(
        _point_kernel,
        grid=(B,),
        in_specs=[
            pl.BlockSpec((1, C, N), lambda b: (b, 0, 0)),
            pl.BlockSpec((C, 2 * C), lambda b: (0, 0)),
            pl.BlockSpec((C, 1), lambda b: (0, 0)),
            pl.BlockSpec((C, KG), lambda b: (0, 0)),
            pl.BlockSpec((KG, 1), lambda b: (0, 0)),
        ],
        out_specs=[
            pl.BlockSpec((1, C, N), lambda b: (b, 0, 0)),
            pl.BlockSpec((1, N, C), lambda b: (b, 0, 0)),
            pl.BlockSpec((1, KG, N), lambda b: (b, 0, 0)),
        ],
        out_shape=[
            jax.ShapeDtypeStruct((B, C, N), jnp.float32),
            jax.ShapeDtypeStruct((B, N, C), jnp.float32),
            jax.ShapeDtypeStruct((B, KG, N), jnp.float32),
        ],
    )(x, w_attn, b_attn, w_aw2, b_aw2)


def _knn_kernel(xall_ref, xblk_ref, idx_ref):
    b = pl.program_id(0)
    xall = xall_ref[0]
    xblk = xblk_ref[0]
    xx = jnp.sum(xall * xall, axis=0, keepdims=True)
    ip = lax.dot_general(xblk, xall, (((0,), (0,)), ((), ())),
                         preferred_element_type=jnp.float32)
    score = 2.0 * ip - xx
    col = lax.broadcasted_iota(jnp.int32, (NB_KNN, N), 1)
    neg = jnp.float32(-jnp.inf)
    picks = []
    for _ in range(K):
        m = jnp.max(score, axis=1, keepdims=True)
        am = jnp.min(jnp.where(score == m, col, N), axis=1,
                     keepdims=True)
        picks.append(am)
        score = jnp.where(col == am, neg, score)
    idx = jnp.concatenate(picks, axis=1)
    idx_ref[0] = idx + b * N


def _knn_call(x):
    return pl.pallas_call(
        _knn_kernel,
        grid=(B, N // NB_KNN),
        in_specs=[
            pl.BlockSpec((1, C, N), lambda b, i: (b, 0, 0)),
            pl.BlockSpec((1, C, NB_KNN), lambda b, i: (b, 0, i)),
        ],
        out_specs=pl.BlockSpec((1, NB_KNN, K), lambda b, i: (b, i, 0)),
        out_shape=jax.ShapeDtypeStruct((B, N, K), jnp.int32),
    )(x, x)


@functools.cache
def _make_gather():
    mesh = plsc.VectorSubcoreMesh(core_axis_name="c", subcore_axis_name="s")

    @functools.partial(
        pl.kernel,
        out_type=jax.ShapeDtypeStruct((B * N * K, C), jnp.float32),
        mesh=mesh,
        scratch_types=[
            pltpu.VMEM((GCHUNK,), jnp.int32),
            pltpu.VMEM((GCHUNK, C), jnp.float32),
            pltpu.SemaphoreType.DMA,
        ],
    )
    def gather_k(table_hbm, idx_hbm, out_hbm, idx_v, rows_v, sem):
        wid = lax.axis_index("s") * 2 + lax.axis_index("c")
        base = wid * ROWS_PER_W
        for ch in range(NCHUNK):
            off = base + ch * GCHUNK
            pltpu.sync_copy(idx_hbm.at[pl.ds(off, GCHUNK)], idx_v)
            pltpu.async_copy(table_hbm.at[idx_v], rows_v, sem).wait()
            pltpu.sync_copy(rows_v, out_hbm.at[pl.ds(off, GCHUNK)])

    return gather_k


def _gather_call(table, idx_flat):
    return _make_gather()(table, idx_flat)


def _attn_kernel(u_ref, x_ref, awp_ref, fv_ref, h_ref):
    u = u_ref[0]
    xb = x_ref[0]
    awp = awp_ref[0]
    fv = fv_ref[0]
    nb = u.shape[1]
    f = []
    e = []
    for k in range(K):
        ft = jnp.transpose(fv[:, k, :])
        fk = u - ft
        f.append(fk)
        e.append(jnp.where(fk > 0, fk, jnp.exp(fk) - 1.0))
    inv = jnp.float32(1.0 / math.sqrt(32.0))
    gm = {}
    for ki in range(K):
        for kj in range(ki, K):
            p = (f[ki] * f[kj]).reshape(G, CG, nb)
            v = jnp.sum(p, axis=1) * inv
            gm[(ki, kj)] = v
            gm[(kj, ki)] = v
    wacc = jnp.zeros((K, G, nb), jnp.float32)
    for ki in range(K):
        rows = jnp.stack([gm[(ki, kj)] for kj in range(K)], axis=0)
        m = jnp.max(rows, axis=0, keepdims=True)
        ex = jnp.exp(rows - m)
        s = ex / jnp.sum(ex, axis=0, keepdims=True)
        awk = awp[ki * G:(ki + 1) * G, :]
        wacc = wacc + s * awk[None]
    lf = jnp.zeros((C, nb), jnp.float32)
    for kj in range(K):
        wrep = jnp.broadcast_to(wacc[kj][:, None, :], (G, CG, nb))
        lf = lf + wrep.reshape(C, nb) * e[kj]
    h_ref[0] = lf + xb


def _attn_call(u, x, awp, fv):
    return pl.pallas_call(
        _attn_kernel,
        grid=(B, N // NB_ATT),
        in_specs=[
            pl.BlockSpec((1, C, NB_ATT), lambda b, i: (b, 0, i)),
            pl.BlockSpec((1, C, NB_ATT), lambda b, i: (b, 0, i)),
            pl.BlockSpec((1, KG, NB_ATT), lambda b, i: (b, 0, i)),
            pl.BlockSpec((1, NB_ATT, K, C), lambda b, i: (b, i, 0, 0)),
        ],
        out_specs=pl.BlockSpec((1, C, NB_ATT), lambda b, i: (b, 0, i)),
        out_shape=jax.ShapeDtypeStruct((B, C, N), jnp.float32),
    )(u, x, awp, fv)


def _norm_conv_kernel(h_ref, gng_ref, gnb_ref, wconv_ref, bconv_ref,
                      bng_ref, bnb_ref, y_ref):
    h = h_ref[0]
    denom = jnp.float32(1.0 / (CG * N))
    parts = []
    for g in range(G):
        hg = h[g * CG:(g + 1) * CG, :]
        mean = jnp.sum(hg, axis=0, keepdims=True)
        mean = jnp.sum(mean, axis=1, keepdims=True) * denom
        d = hg - mean
        var = jnp.sum(d * d, axis=0, keepdims=True)
        var = jnp.sum(var, axis=1, keepdims=True) * denom
        parts.append(d * lax.rsqrt(var + 1e-5))
    hn = jnp.concatenate(parts, axis=0)
    hn = hn * gng_ref[:] + gnb_ref[:]
    y = lax.dot_general(wconv_ref[:], hn, (((1,), (0,)), ((), ())),
                        preferred_element_type=jnp.float32)
    y = y + bconv_ref[:]
    scale = bng_ref[:] * lax.rsqrt(jnp.float32(1.0 + 1e-5))
    y = y * scale + bnb_ref[:]
    y_ref[0] = jnp.maximum(y, 0.0)


def _norm_conv_call(h, gn_gamma, gn_beta, w_conv, b_conv, bn_gamma, bn_beta):
    return pl.pallas_call(
        _norm_conv_kernel,
        grid=(B,),
        in_specs=[
            pl.BlockSpec((1, C, N), lambda b: (b, 0, 0)),
            pl.BlockSpec((C, 1), lambda b: (0, 0)),
            pl.BlockSpec((C, 1), lambda b: (0, 0)),
            pl.BlockSpec((C, C), lambda b: (0, 0)),
            pl.BlockSpec((C, 1), lambda b: (0, 0)),
            pl.BlockSpec((C, 1), lambda b: (0, 0)),
            pl.BlockSpec((C, 1), lambda b: (0, 0)),
        ],
        out_specs=pl.BlockSpec((1, C, N), lambda b: (b, 0, 0)),
        out_shape=jax.ShapeDtypeStruct((B, C, N), jnp.float32),
    )(h, gn_gamma, gn_beta, w_conv, b_conv, bn_gamma, bn_beta)


def kernel(features, W_attn, b_attn, gn_gamma, gn_beta, W_conv, b_conv,
           bn_gamma, bn_beta, W_aw, b_aw):
    x = features.reshape(B, C, N)
    j = jnp.arange(KG)
    perm = (j % G) * K + j // G
    w_aw2 = W_aw[:, perm]
    b_aw2 = b_aw[perm]
    u, vt, awp = _point_call(x, W_attn, b_attn.reshape(C, 1), w_aw2,
                             b_aw2.reshape(KG, 1))
    idx = _knn_call(x)
    table = vt.reshape(B * N, C)
    fv = _gather_call(table, idx.reshape(B * N * K))
    h = _attn_call(u, x, awp, fv.reshape(B, N, K, C))
    y = _norm_conv_call(h, gn_gamma.reshape(C, 1), gn_beta.reshape(C, 1),
                        W_conv, b_conv.reshape(C, 1), bn_gamma.reshape(C, 1),
                        bn_beta.reshape(C, 1))
    return y.reshape(B, C, N, 1)


# speedup vs baseline: 1.1731x; 1.1731x over previous
import functools
import math

import jax
import jax.numpy as jnp
from jax import lax
from jax.experimental import pallas as pl
from jax.experimental.pallas import tpu as pltpu
from jax.experimental.pallas import tpu_sc as plsc

B, C, N, K, G = 2, 128, 4096, 9, 4
CG = C // G
KG = K * G
NB_KNN = 256
NB_ATT = 512
NW = 32
NK = N * K
ROWS_PER_W = NK // NW
GCHUNK = 128
NCHUNK = ROWS_PER_W // GCHUNK


def _point_kernel(x_ref, wattn_ref, battn_ref, waw2_ref, baw2_ref,
                  u_ref, vt_ref, awp_ref):
    x = x_ref[0]
    w1 = wattn_ref[:, :C]
    w2 = wattn_ref[:, C:]
    wu = w1 + w2
    u_ref[0] = lax.dot_general(wu, x, (((1,), (0,)), ((), ())),
                               preferred_element_type=jnp.float32) \
        + battn_ref[:]
    vt_ref[0] = lax.dot_general(x, w2, (((0,), (1,)), ((), ())),
                                preferred_element_type=jnp.float32)
    awl = lax.dot_general(waw2_ref[:], x, (((0,), (0,)), ((), ())),
                          preferred_element_type=jnp.float32)
    awl = awl + baw2_ref[:]
    a3 = awl.reshape(K, G, N)
    m = jnp.max(a3, axis=0, keepdims=True)
    e = jnp.exp(a3 - m)
    p = e / jnp.sum(e, axis=0, keepdims=True)
    awp_ref[0] = p.reshape(KG, N)


def _point_call(x, w_attn, b_attn, w_aw2, b_aw2):
    return pl.pallas_call(
        _point_kernel,
        grid=(B,),
        in_specs=[
            pl.BlockSpec((1, C, N), lambda b: (b, 0, 0)),
            pl.BlockSpec((C, 2 * C), lambda b: (0, 0)),
            pl.BlockSpec((C, 1), lambda b: (0, 0)),
            pl.BlockSpec((C, KG), lambda b: (0, 0)),
            pl.BlockSpec((KG, 1), lambda b: (0, 0)),
        ],
        out_specs=[
            pl.BlockSpec((1, C, N), lambda b: (b, 0, 0)),
            pl.BlockSpec((1, N, C), lambda b: (b, 0, 0)),
            pl.BlockSpec((1, KG, N), lambda b: (b, 0, 0)),
        ],
        out_shape=[
            jax.ShapeDtypeStruct((B, C, N), jnp.float32),
            jax.ShapeDtypeStruct((B, N, C), jnp.float32),
            jax.ShapeDtypeStruct((B, KG, N), jnp.float32),
        ],
    )(x, w_attn, b_attn, w_aw2, b_aw2)


def _knn_kernel(xall_ref, xblk_ref, idx_ref, *, boff):
    xall = xall_ref[0]
    xblk = xblk_ref[0]
    xx = jnp.sum(xall * xall, axis=0, keepdims=True)
    ip = lax.dot_general(xblk, xall, (((0,), (0,)), ((), ())),
                         preferred_element_type=jnp.float32)
    score = 2.0 * ip - xx
    nch = N // 128
    lane = lax.broadcasted_iota(jnp.int32, (NB_KNN, 128), 1)
    neg = jnp.float32(-jnp.inf)
    big = jnp.int32(N)
    picks = []
    am = None
    for r in range(K):
        chunks = []
        val = None
        cid = None
        for c in range(nch):
            sc = score[:, c * 128:(c + 1) * 128]
            if am is not None:
                sc = jnp.where(lane + c * 128 == am, neg, sc)
                chunks.append(sc)
            if val is None:
                val = sc
                cid = jnp.zeros_like(lane)
            else:
                gt = sc > val
                val = jnp.where(gt, sc, val)
                cid = jnp.where(gt, jnp.int32(c), cid)
        if chunks and r < K - 1:
            score = jnp.concatenate(chunks, axis=1)
        colv = cid * 128 + lane
        m = jnp.max(val, axis=1, keepdims=True)
        am = jnp.min(jnp.where(val == m, colv, big), axis=1,
                     keepdims=True)
        picks.append(am)
    idx = jnp.concatenate(picks, axis=1)
    idx_ref[0] = idx + boff


def _knn_call(x, b):
    return pl.pallas_call(
        functools.partial(_knn_kernel, boff=b * N),
        grid=(N // NB_KNN,),
        in_specs=[
            pl.BlockSpec((1, C, N), lambda i, b=b: (b, 0, 0)),
            pl.BlockSpec((1, C, NB_KNN), lambda i, b=b: (b, 0, i)),
        ],
        out_specs=pl.BlockSpec((1, NB_KNN, K), lambda i: (0, i, 0)),
        out_shape=jax.ShapeDtypeStruct((1, N, K), jnp.int32),
    )(x, x)


@functools.cache
def _make_gather():
    mesh = plsc.VectorSubcoreMesh(core_axis_name="c", subcore_axis_name="s")

    @functools.partial(
        pl.kernel,
        out_type=jax.ShapeDtypeStruct((NK, C), jnp.float32),
        mesh=mesh,
        scratch_types=[
            pltpu.VMEM((GCHUNK,), jnp.int32),
            pltpu.VMEM((GCHUNK, C), jnp.float32),
            pltpu.SemaphoreType.DMA,
        ],
    )
    def gather_k(table_hbm, idx_hbm, out_hbm, idx_v, rows_v, sem):
        wid = lax.axis_index("s") * 2 + lax.axis_index("c")
        base = wid * ROWS_PER_W
        for ch in range(NCHUNK):
            off = base + ch * GCHUNK
            pltpu.sync_copy(idx_hbm.at[pl.ds(off, GCHUNK)], idx_v)
            pltpu.async_copy(table_hbm.at[idx_v], rows_v, sem).wait()
            pltpu.sync_copy(rows_v, out_hbm.at[pl.ds(off, GCHUNK)])

    return gather_k


def _gather_call(table, idx_flat):
    return _make_gather()(table, idx_flat)


def _attn_kernel(u_ref, x_ref, awp_ref, fv_ref, h_ref):
    u = u_ref[0]
    xb = x_ref[0]
    awp = awp_ref[0]
    fv = fv_ref[0]
    nb = u.shape[1]
    f = []
    e = []
    for k in range(K):
        ft = jnp.transpose(fv[:, k, :])
        fk = u - ft
        f.append(fk)
        e.append(jnp.where(fk > 0, fk, jnp.exp(fk) - 1.0))
    inv = jnp.float32(1.0 / math.sqrt(32.0))
    gm = {}
    for ki in range(K):
        for kj in range(ki, K):
            p = (f[ki] * f[kj]).reshape(G, CG, nb)
            v = jnp.sum(p, axis=1) * inv
            gm[(ki, kj)] = v
            gm[(kj, ki)] = v
    wacc = jnp.zeros((K, G, nb), jnp.float32)
    for ki in range(K):
        rows = jnp.stack([gm[(ki, kj)] for kj in range(K)], axis=0)
        m = jnp.max(rows, axis=0, keepdims=True)
        ex = jnp.exp(rows - m)
        s = ex / jnp.sum(ex, axis=0, keepdims=True)
        awk = awp[ki * G:(ki + 1) * G, :]
        wacc = wacc + s * awk[None]
    lf = jnp.zeros((C, nb), jnp.float32)
    for kj in range(K):
        wrep = jnp.broadcast_to(wacc[kj][:, None, :], (G, CG, nb))
        lf = lf + wrep.reshape(C, nb) * e[kj]
    h_ref[0] = lf + xb


def _attn_call(u, x, awp, fv, b):
    return pl.pallas_call(
        _attn_kernel,
        grid=(N // NB_ATT,),
        in_specs=[
            pl.BlockSpec((1, C, NB_ATT), lambda i, b=b: (b, 0, i)),
            pl.BlockSpec((1, C, NB_ATT), lambda i, b=b: (b, 0, i)),
            pl.BlockSpec((1, KG, NB_ATT), lambda i, b=b: (b, 0, i)),
            pl.BlockSpec((1, NB_ATT, K, C), lambda i: (0, i, 0, 0)),
        ],
        out_specs=pl.BlockSpec((1, C, NB_ATT), lambda i: (0, 0, i)),
        out_shape=jax.ShapeDtypeStruct((1, C, N), jnp.float32),
    )(u, x, awp, fv)


def _norm_conv_kernel(h_ref, gng_ref, gnb_ref, wconv_ref, bconv_ref,
                      bng_ref, bnb_ref, y_ref):
    h = h_ref[0]
    denom = jnp.float32(1.0 / (CG * N))
    parts = []
    for g in range(G):
        hg = h[g * CG:(g + 1) * CG, :]
        mean = jnp.sum(hg, axis=0, keepdims=True)
        mean = jnp.sum(mean, axis=1, keepdims=True) * denom
        d = hg - mean
        var = jnp.sum(d * d, axis=0, keepdims=True)
        var = jnp.sum(var, axis=1, keepdims=True) * denom
        parts.append(d * lax.rsqrt(var + 1e-5))
    hn = jnp.concatenate(parts, axis=0)
    hn = hn * gng_ref[:] + gnb_ref[:]
    y = lax.dot_general(wconv_ref[:], hn, (((1,), (0,)), ((), ())),
                        preferred_element_type=jnp.float32)
    y = y + bconv_ref[:]
    scale = bng_ref[:] * lax.rsqrt(jnp.float32(1.0 + 1e-5))
    y = y * scale + bnb_ref[:]
    y_ref[0] = jnp.maximum(y, 0.0)


def _norm_conv_call(h, gn_gamma, gn_beta, w_conv, b_conv, bn_gamma, bn_beta):
    return pl.pallas_call(
        _norm_conv_kernel,
        grid=(1,),
        in_specs=[
            pl.BlockSpec((1, C, N), lambda b: (0, 0, 0)),
            pl.BlockSpec((C, 1), lambda b: (0, 0)),
            pl.BlockSpec((C, 1), lambda b: (0, 0)),
            pl.BlockSpec((C, C), lambda b: (0, 0)),
            pl.BlockSpec((C, 1), lambda b: (0, 0)),
            pl.BlockSpec((C, 1), lambda b: (0, 0)),
            pl.BlockSpec((C, 1), lambda b: (0, 0)),
        ],
        out_specs=pl.BlockSpec((1, C, N), lambda b: (0, 0, 0)),
        out_shape=jax.ShapeDtypeStruct((1, C, N), jnp.float32),
    )(h, gn_gamma, gn_beta, w_conv, b_conv, bn_gamma, bn_beta)


def kernel(features, W_attn, b_attn, gn_gamma, gn_beta, W_conv, b_conv,
           bn_gamma, bn_beta, W_aw, b_aw):
    x = features.reshape(B, C, N)
    j = jnp.arange(KG)
    perm = (j % G) * K + j // G
    w_aw2 = W_aw[:, perm]
    b_aw2 = b_aw[perm]
    u, vt, awp = _point_call(x, W_attn, b_attn.reshape(C, 1), w_aw2,
                             b_aw2.reshape(KG, 1))
    table = vt.reshape(B * N, C)
    fvs = [
        _gather_call(table, _knn_call(x, b).reshape(NK)) for b in range(B)
    ]
    ys = []
    for b in range(B):
        h = _attn_call(u, x, awp, fvs[b].reshape(1, N, K, C), b)
        ys.append(_norm_conv_call(
            h, gn_gamma.reshape(C, 1), gn_beta.reshape(C, 1),
            W_conv, b_conv.reshape(C, 1), bn_gamma.reshape(C, 1),
            bn_beta.reshape(C, 1)))
    return jnp.concatenate(ys, axis=0).reshape(B, C, N, 1)


# speedup vs baseline: 1.2367x; 1.0542x over previous
import functools
import math

import jax
import jax.numpy as jnp
from jax import lax
from jax.experimental import pallas as pl
from jax.experimental.pallas import tpu as pltpu
from jax.experimental.pallas import tpu_sc as plsc

B, C, N, K, G = 2, 128, 4096, 9, 4
CG = C // G
KG = K * G
NB_KNN = 512
NB_ATT = 512
NW = 32
NK = N * K
ROWS_PER_W = NK // NW
GCHUNK = 128
NCHUNK = ROWS_PER_W // GCHUNK


def _point_kernel(x_ref, wattn_ref, battn_ref, waw2_ref, baw2_ref,
                  u_ref, vt_ref, awp_ref):
    x = x_ref[0]
    w1 = wattn_ref[:, :C]
    w2 = wattn_ref[:, C:]
    wu = w1 + w2
    u_ref[0] = lax.dot_general(wu, x, (((1,), (0,)), ((), ())),
                               preferred_element_type=jnp.float32) \
        + battn_ref[:]
    vt_ref[0] = lax.dot_general(x, w2, (((0,), (1,)), ((), ())),
                                preferred_element_type=jnp.float32)
    awl = lax.dot_general(waw2_ref[:], x, (((0,), (0,)), ((), ())),
                          preferred_element_type=jnp.float32)
    awl = awl + baw2_ref[:]
    a3 = awl.reshape(K, G, N)
    m = jnp.max(a3, axis=0, keepdims=True)
    e = jnp.exp(a3 - m)
    p = e / jnp.sum(e, axis=0, keepdims=True)
    awp_ref[0] = p.reshape(KG, N)


def _point_call(x, w_attn, b_attn, w_aw2, b_aw2):
    return pl.pallas_call(
        _point_kernel,
        grid=(B,),
        in_specs=[
            pl.BlockSpec((1, C, N), lambda b: (b, 0, 0)),
            pl.BlockSpec((C, 2 * C), lambda b: (0, 0)),
            pl.BlockSpec((C, 1), lambda b: (0, 0)),
            pl.BlockSpec((C, KG), lambda b: (0, 0)),
            pl.BlockSpec((KG, 1), lambda b: (0, 0)),
        ],
        out_specs=[
            pl.BlockSpec((1, C, N), lambda b: (b, 0, 0)),
            pl.BlockSpec((1, N, C), lambda b: (b, 0, 0)),
            pl.BlockSpec((1, KG, N), lambda b: (b, 0, 0)),
        ],
        out_shape=[
            jax.ShapeDtypeStruct((B, C, N), jnp.float32),
            jax.ShapeDtypeStruct((B, N, C), jnp.float32),
            jax.ShapeDtypeStruct((B, KG, N), jnp.float32),
        ],
    )(x, w_attn, b_attn, w_aw2, b_aw2)


def _knn_kernel(xall_ref, xblk_ref, idx_ref, *, boff):
    xall = xall_ref[0]
    xblk = xblk_ref[0]
    xx = jnp.sum(xall * xall, axis=0, keepdims=True)
    ip = lax.dot_general(xblk, xall, (((0,), (0,)), ((), ())),
                         preferred_element_type=jnp.float32)
    score = 2.0 * ip - xx
    nch = N // 128
    lane = lax.broadcasted_iota(jnp.int32, (NB_KNN, 128), 1)
    neg = jnp.float32(-jnp.inf)
    big = jnp.int32(N)
    picks = []
    am = None
    for r in range(K):
        chunks = []
        val = None
        cid = None
        for c in range(nch):
            sc = score[:, c * 128:(c + 1) * 128]
            if am is not None:
                sc = jnp.where(lane + c * 128 == am, neg, sc)
                chunks.append(sc)
            if val is None:
                val = sc
                cid = jnp.zeros_like(lane)
            else:
                gt = sc > val
                val = jnp.where(gt, sc, val)
                cid = jnp.where(gt, jnp.int32(c), cid)
        if chunks and r < K - 1:
            score = jnp.concatenate(chunks, axis=1)
        colv = cid * 128 + lane
        m = jnp.max(val, axis=1, keepdims=True)
        am = jnp.min(jnp.where(val == m, colv, big), axis=1,
                     keepdims=True)
        picks.append(am)
    idx = jnp.concatenate(picks, axis=1)
    idx_ref[0] = idx + boff


def _knn_call(x, b):
    return pl.pallas_call(
        functools.partial(_knn_kernel, boff=b * N),
        grid=(N // NB_KNN,),
        in_specs=[
            pl.BlockSpec((1, C, N), lambda i, b=b: (b, 0, 0)),
            pl.BlockSpec((1, C, NB_KNN), lambda i, b=b: (b, 0, i)),
        ],
        out_specs=pl.BlockSpec((1, NB_KNN, K), lambda i: (0, i, 0)),
        out_shape=jax.ShapeDtypeStruct((1, N, K), jnp.int32),
    )(x, x)


@functools.cache
def _make_gather():
    mesh = plsc.VectorSubcoreMesh(core_axis_name="c", subcore_axis_name="s")

    @functools.partial(
        pl.kernel,
        out_type=jax.ShapeDtypeStruct((NK, C), jnp.float32),
        mesh=mesh,
        scratch_types=[
            pltpu.VMEM((GCHUNK,), jnp.int32),
            pltpu.VMEM((GCHUNK, C), jnp.float32),
            pltpu.SemaphoreType.DMA,
        ],
    )
    def gather_k(table_hbm, idx_hbm, out_hbm, idx_v, rows_v, sem):
        wid = lax.axis_index("s") * 2 + lax.axis_index("c")
        base = wid * ROWS_PER_W
        for ch in range(NCHUNK):
            off = base + ch * GCHUNK
            pltpu.sync_copy(idx_hbm.at[pl.ds(off, GCHUNK)], idx_v)
            pltpu.async_copy(table_hbm.at[idx_v], rows_v, sem).wait()
            pltpu.sync_copy(rows_v, out_hbm.at[pl.ds(off, GCHUNK)])

    return gather_k


def _gather_call(table, idx_flat):
    return _make_gather()(table, idx_flat)


def _attn_kernel(u_ref, x_ref, awp_ref, fv_ref, h_ref):
    u = u_ref[0]
    xb = x_ref[0]
    awp = awp_ref[0]
    fv = fv_ref[0]
    nb = u.shape[1]
    f = []
    e = []
    for k in range(K):
        ft = jnp.transpose(fv[:, k, :])
        fk = u - ft
        f.append(fk)
        e.append(jnp.where(fk > 0, fk, jnp.exp(fk) - 1.0))
    inv = jnp.float32(1.0 / math.sqrt(32.0))
    gm = {}
    for ki in range(K):
        for kj in range(ki, K):
            p = (f[ki] * f[kj]).reshape(G, CG, nb)
            v = jnp.sum(p, axis=1) * inv
            gm[(ki, kj)] = v
            gm[(kj, ki)] = v
    wacc = jnp.zeros((K, G, nb), jnp.float32)
    for ki in range(K):
        rows = jnp.stack([gm[(ki, kj)] for kj in range(K)], axis=0)
        m = jnp.max(rows, axis=0, keepdims=True)
        ex = jnp.exp(rows - m)
        s = ex / jnp.sum(ex, axis=0, keepdims=True)
        awk = awp[ki * G:(ki + 1) * G, :]
        wacc = wacc + s * awk[None]
    lf = jnp.zeros((C, nb), jnp.float32)
    for kj in range(K):
        wrep = jnp.broadcast_to(wacc[kj][:, None, :], (G, CG, nb))
        lf = lf + wrep.reshape(C, nb) * e[kj]
    h_ref[0] = lf + xb


def _attn_call(u, x, awp, fv, b):
    return pl.pallas_call(
        _attn_kernel,
        grid=(N // NB_ATT,),
        in_specs=[
            pl.BlockSpec((1, C, NB_ATT), lambda i, b=b: (b, 0, i)),
            pl.BlockSpec((1, C, NB_ATT), lambda i, b=b: (b, 0, i)),
            pl.BlockSpec((1, KG, NB_ATT), lambda i, b=b: (b, 0, i)),
            pl.BlockSpec((1, NB_ATT, K, C), lambda i: (0, i, 0, 0)),
        ],
        out_specs=pl.BlockSpec((1, C, NB_ATT), lambda i: (0, 0, i)),
        out_shape=jax.ShapeDtypeStruct((1, C, N), jnp.float32),
    )(u, x, awp, fv)


def _norm_conv_kernel(h_ref, gng_ref, gnb_ref, wconv_ref, bconv_ref,
                      bng_ref, bnb_ref, y_ref):
    h = h_ref[0]
    denom = jnp.float32(1.0 / (CG * N))
    parts = []
    for g in range(G):
        hg = h[g * CG:(g + 1) * CG, :]
        mean = jnp.sum(hg, axis=0, keepdims=True)
        mean = jnp.sum(mean, axis=1, keepdims=True) * denom
        d = hg - mean
        var = jnp.sum(d * d, axis=0, keepdims=True)
        var = jnp.sum(var, axis=1, keepdims=True) * denom
        parts.append(d * lax.rsqrt(var + 1e-5))
    hn = jnp.concatenate(parts, axis=0)
    hn = hn * gng_ref[:] + gnb_ref[:]
    y = lax.dot_general(wconv_ref[:], hn, (((1,), (0,)), ((), ())),
                        preferred_element_type=jnp.float32)
    y = y + bconv_ref[:]
    scale = bng_ref[:] * lax.rsqrt(jnp.float32(1.0 + 1e-5))
    y = y * scale + bnb_ref[:]
    y_ref[0] = jnp.maximum(y, 0.0)


def _norm_conv_call(h, gn_gamma, gn_beta, w_conv, b_conv, bn_gamma, bn_beta):
    return pl.pallas_call(
        _norm_conv_kernel,
        grid=(1,),
        in_specs=[
            pl.BlockSpec((1, C, N), lambda b: (0, 0, 0)),
            pl.BlockSpec((C, 1), lambda b: (0, 0)),
            pl.BlockSpec((C, 1), lambda b: (0, 0)),
            pl.BlockSpec((C, C), lambda b: (0, 0)),
            pl.BlockSpec((C, 1), lambda b: (0, 0)),
            pl.BlockSpec((C, 1), lambda b: (0, 0)),
            pl.BlockSpec((C, 1), lambda b: (0, 0)),
        ],
        out_specs=pl.BlockSpec((1, C, N), lambda b: (0, 0, 0)),
        out_shape=jax.ShapeDtypeStruct((1, C, N), jnp.float32),
    )(h, gn_gamma, gn_beta, w_conv, b_conv, bn_gamma, bn_beta)


def kernel(features, W_attn, b_attn, gn_gamma, gn_beta, W_conv, b_conv,
           bn_gamma, bn_beta, W_aw, b_aw):
    x = features.reshape(B, C, N)
    j = jnp.arange(KG)
    perm = (j % G) * K + j // G
    w_aw2 = W_aw[:, perm]
    b_aw2 = b_aw[perm]
    u, vt, awp = _point_call(x, W_attn, b_attn.reshape(C, 1), w_aw2,
                             b_aw2.reshape(KG, 1))
    table = vt.reshape(B * N, C)
    fvs = [
        _gather_call(table, _knn_call(x, b).reshape(NK)) for b in range(B)
    ]
    ys = []
    for b in range(B):
        h = _attn_call(u, x, awp, fvs[b].reshape(1, N, K, C), b)
        ys.append(_norm_conv_call(
            h, gn_gamma.reshape(C, 1), gn_beta.reshape(C, 1),
            W_conv, b_conv.reshape(C, 1), bn_gamma.reshape(C, 1),
            bn_beta.reshape(C, 1)))
    return jnp.concatenate(ys, axis=0).reshape(B, C, N, 1)
